# baseline (speedup 1.0000x reference)
import functools

import jax
import jax.numpy as jnp
from jax import lax
from jax.experimental import pallas as pl
from jax.experimental.pallas import tpu as pltpu
from jax.experimental.pallas import tpu_sc as plsc

NUM_CORES = 2
NUM_SUBCORES = 16
NUM_WORKERS = NUM_CORES * NUM_SUBCORES
LANES = 16
PAD_IDX = 0
NBUF = 4
NOBUF = 2


def _make_lookup(b, l, d, v):
    assert b % (NUM_WORKERS * 8) == 0
    bw = b // NUM_WORKERS
    assert bw == 128
    assert d == 64
    assert l % NBUF == 0
    jgroups = bw // LANES

    mesh = plsc.VectorSubcoreMesh(core_axis_name="c", subcore_axis_name="s")

    @functools.partial(
        pl.kernel,
        mesh=mesh,
        compiler_params=pltpu.CompilerParams(needs_layout_passes=False),
        out_type=jax.ShapeDtypeStruct((l, d, b), jnp.float32),
        scratch_types=[
            pltpu.VMEM((NBUF, bw), jnp.int32),
            pltpu.VMEM((NBUF, bw, 2 * d), jnp.float32),
            pltpu.VMEM((NOBUF, d, bw), jnp.float32),
            pltpu.VMEM((l, d), jnp.float32),
            pltpu.VMEM((NBUF, bw), jnp.int32),
            pltpu.VMEM((bw * d,), jnp.float32),
            pltpu.SemaphoreType.DMA((NBUF,)),
            pltpu.SemaphoreType.DMA((NBUF,)),
            pltpu.SemaphoreType.DMA((NOBUF,)),
        ],
    )
    def body(xt_hbm, table2_hbm, pe_hbm, out_hbm, idxr, gbuf, obuf, pe_v,
             gidx, skew, isem, gsem, ssem):
        cid = lax.axis_index("c")
        sid = lax.axis_index("s")
        wid = sid * NUM_CORES + cid
        wcol = wid * bw

        pltpu.sync_copy(pe_hbm, pe_v)

        iota = lax.iota(jnp.int32, LANES)

        def idx_copy(li, ib):
            return pltpu.make_async_copy(
                xt_hbm.at[li, pl.ds(wcol, bw)], idxr.at[ib], isem.at[ib]
            )

        def prep_and_start_gather(gb):
            for j in range(jgroups):
                vv = idxr[gb, pl.ds(j * LANES, LANES)]
                gidx[gb, pl.ds(j * LANES, LANES)] = lax.shift_right_logical(vv, 1)
            pltpu.make_async_copy(
                table2_hbm.at[gidx.at[gb]], gbuf.at[gb], gsem.at[gb]
            ).start()

        def wait_gather(gb):
            pltpu.make_async_copy(
                table2_hbm.at[gidx.at[gb]], gbuf.at[gb], gsem.at[gb]
            ).wait()

        def store_copy(li, ob):
            return pltpu.make_async_copy(
                obuf.at[ob], out_hbm.at[li, :, pl.ds(wcol, bw)], ssem.at[ob]
            )

        def compute(li, gb, ob):
            iota128 = iota * 128
            pevs = tuple(pe_v[li, pl.ds(gg * LANES, LANES)]
                         for gg in range(d // LANES))

            @plsc.parallel_loop(0, jgroups, carry=pevs)
            def jg_loop(jg, pev):
                j0 = jg * LANES
                vv = idxr[gb, pl.ds(j0, LANES)]
                cols = lax.shift_left(vv & 1, 6)
                fact = jnp.where(vv != PAD_IDX, 1.0, 0.0)
                for m in range(LANES):
                    off = cols[m]
                    fa = fact[m]
                    j = j0 + m
                    patt = iota128 + lax.shift_left((iota + m) & 15, 3)
                    for gg in range(d // LANES):
                        g = gbuf[gb, j, pl.ds(off + 16 * gg, LANES)]
                        g = g * fa + pev[gg]
                        plsc.store_scatter(
                            skew, [patt + (16 * gg * 128) + jg], g
                        )
                return pev

            @plsc.parallel_loop(0, d, unroll=2)
            def f_loop(f):
                rot = lax.shift_left((iota + f) & 15, 3)
                base = f * 128
                for jg in range(jgroups):
                    g = plsc.load_gather(skew, [rot + (base + jg)])
                    obuf[ob, f, pl.ds(jg * LANES, LANES)] = g

        for li0 in range(3):
            idx_copy(jnp.int32(li0), li0).start()
        for li0 in range(2):
            idx_copy(jnp.int32(li0), li0).wait()
            prep_and_start_gather(li0)

        def quad(t, carry):
            for bb in range(NBUF):
                li = t * NBUF + bb
                ob = bb % NOBUF

                @pl.when(li + 3 < l)
                def _refill_idx():
                    idx_copy(li + 3, (bb + 3) % NBUF).start()

                @pl.when(li + 2 < l)
                def _refill_gather():
                    idx_copy(li + 2, (bb + 2) % NBUF).wait()
                    prep_and_start_gather((bb + 2) % NBUF)

                wait_gather(bb)

                @pl.when(li >= NOBUF)
                def _drain():
                    store_copy(li - NOBUF, ob).wait()

                compute(li, bb, ob)
                store_copy(li, ob).start()
            return carry

        lax.fori_loop(0, l // NBUF, quad, 0)

        for li0 in range(l - NOBUF, l):
            store_copy(jnp.int32(li0), li0 % NOBUF).wait()

    return body


def kernel(x, table, pe):
    b, l = x.shape
    v, d = table.shape
    xt = x.T
    table2 = table.reshape(v // 2, 2 * d)
    pe_l = pe[:l]
    lookup = _make_lookup(b, l, d, v)
    out_t = lookup(xt, table2, pe_l)
    return out_t.transpose(2, 0, 1)


# speedup vs baseline: 1.0805x; 1.0805x over previous
import functools

import jax
import jax.numpy as jnp
from jax import lax
from jax.experimental import pallas as pl
from jax.experimental.pallas import tpu as pltpu
from jax.experimental.pallas import tpu_sc as plsc

NUM_CORES = 2
NUM_SUBCORES = 16
NUM_WORKERS = NUM_CORES * NUM_SUBCORES
LANES = 16
PAD_IDX = 0
NBUF = 4
NOBUF = 2


def _make_lookup(b, l, d, v):
    assert b % (NUM_WORKERS * 8) == 0
    bw = b // NUM_WORKERS
    assert bw == 128
    assert d == 64
    assert l % NBUF == 0
    jgroups = bw // LANES

    mesh = plsc.VectorSubcoreMesh(core_axis_name="c", subcore_axis_name="s")

    @functools.partial(
        pl.kernel,
        mesh=mesh,
        compiler_params=pltpu.CompilerParams(needs_layout_passes=False),
        out_type=jax.ShapeDtypeStruct((l, d, b), jnp.float32),
        scratch_types=[
            pltpu.VMEM((NBUF, bw), jnp.int32),
            pltpu.VMEM((NBUF, bw, 2 * d), jnp.float32),
            pltpu.VMEM((NOBUF, d, bw), jnp.float32),
            pltpu.VMEM((l, d), jnp.float32),
            pltpu.VMEM((NBUF, bw), jnp.int32),
            pltpu.VMEM((bw * d,), jnp.float32),
            pltpu.VMEM((bw,), jnp.int32),
            pltpu.SemaphoreType.DMA((NBUF,)),
            pltpu.SemaphoreType.DMA((NBUF,)),
            pltpu.SemaphoreType.DMA((NOBUF,)),
        ],
    )
    def body(xt_hbm, table2_hbm, pe_hbm, out_hbm, idxr, gbuf, obuf, pe_v,
             gidx, skew, cbuf, isem, gsem, ssem):
        cid = lax.axis_index("c")
        sid = lax.axis_index("s")
        wid = sid * NUM_CORES + cid
        wcol = wid * bw

        pltpu.sync_copy(pe_hbm, pe_v)

        iota = lax.iota(jnp.int32, LANES)

        def idx_copy(li, ib):
            return pltpu.make_async_copy(
                xt_hbm.at[li, pl.ds(wcol, bw)], idxr.at[ib], isem.at[ib]
            )

        def prep_and_start_gather(gb):
            for j in range(jgroups):
                vv = idxr[gb, pl.ds(j * LANES, LANES)]
                gidx[gb, pl.ds(j * LANES, LANES)] = lax.shift_right_logical(vv, 1)
            pltpu.make_async_copy(
                table2_hbm.at[gidx.at[gb]], gbuf.at[gb], gsem.at[gb]
            ).start()

        def wait_gather(gb):
            pltpu.make_async_copy(
                table2_hbm.at[gidx.at[gb]], gbuf.at[gb], gsem.at[gb]
            ).wait()

        def store_copy(li, ob):
            return pltpu.make_async_copy(
                obuf.at[ob], out_hbm.at[li, :, pl.ds(wcol, bw)], ssem.at[ob]
            )

        def compute(li, gb, ob):
            iota128 = iota * 128
            iota_g = [iota + (16 * gg) for gg in range(d // LANES)]
            pevs = tuple(pe_v[li, pl.ds(gg * LANES, LANES)]
                         for gg in range(d // LANES))
            masks = []
            for jg in range(jgroups):
                vv = idxr[gb, pl.ds(jg * LANES, LANES)]
                cbuf[pl.ds(jg * LANES, LANES)] = lax.shift_left(vv & 1, 6)
                masks.append(vv != PAD_IDX)
            zero = jnp.zeros((LANES,), jnp.float32)

            for jg in range(jgroups):
                j0 = jg * LANES

                @plsc.parallel_loop(0, LANES, unroll=2)
                def m_loop(m):
                    j_full = jnp.full((LANES,), j0 + m, jnp.int32)
                    off = plsc.load_gather(cbuf, [j_full])
                    patt = iota128 + lax.shift_left((iota + m) & 15, 3)
                    for gg in range(d // LANES):
                        colv = off + iota_g[gg]
                        g = plsc.load_gather(gbuf.at[gb], [j_full, colv])
                        plsc.store_scatter(
                            skew, [patt + (16 * gg * 128 + jg)], g
                        )

            l_full = jnp.full((LANES,), li, jnp.int32)

            @plsc.parallel_loop(0, d, unroll=2, carry=tuple(masks))
            def f_loop(f, c):
                f_full = jnp.full((LANES,), f, jnp.int32)
                pe_b = plsc.load_gather(pe_v, [l_full, f_full])
                rot = lax.shift_left((iota + f) & 15, 3)
                base = f * 128
                for jg in range(jgroups):
                    g = plsc.load_gather(skew, [rot + (base + jg)])
                    g = jnp.where(c[jg], g, zero)
                    obuf[ob, f, pl.ds(jg * LANES, LANES)] = g + pe_b
                return c

        for li0 in range(3):
            idx_copy(jnp.int32(li0), li0).start()
        for li0 in range(2):
            idx_copy(jnp.int32(li0), li0).wait()
            prep_and_start_gather(li0)

        def quad(t, carry):
            for bb in range(NBUF):
                li = t * NBUF + bb
                ob = bb % NOBUF

                @pl.when(li + 3 < l)
                def _refill_idx():
                    idx_copy(li + 3, (bb + 3) % NBUF).start()

                @pl.when(li + 2 < l)
                def _refill_gather():
                    idx_copy(li + 2, (bb + 2) % NBUF).wait()
                    prep_and_start_gather((bb + 2) % NBUF)

                wait_gather(bb)

                @pl.when(li >= NOBUF)
                def _drain():
                    store_copy(li - NOBUF, ob).wait()

                compute(li, bb, ob)
                store_copy(li, ob).start()
            return carry

        lax.fori_loop(0, l // NBUF, quad, 0)

        for li0 in range(l - NOBUF, l):
            store_copy(jnp.int32(li0), li0 % NOBUF).wait()

    return body


def kernel(x, table, pe):
    b, l = x.shape
    v, d = table.shape
    xt = x.T
    table2 = table.reshape(v // 2, 2 * d)
    pe_l = pe[:l]
    lookup = _make_lookup(b, l, d, v)
    out_t = lookup(xt, table2, pe_l)
    return out_t.transpose(2, 0, 1)


# speedup vs baseline: 1.1563x; 1.0702x over previous
import functools

import jax
import jax.numpy as jnp
from jax import lax
from jax.experimental import pallas as pl
from jax.experimental.pallas import tpu as pltpu
from jax.experimental.pallas import tpu_sc as plsc

NUM_CORES = 2
NUM_SUBCORES = 16
NUM_WORKERS = NUM_CORES * NUM_SUBCORES
LANES = 16
PAD_IDX = 0
NBUF = 4
NOBUF = 2


def _make_lookup(b, l, d, v):
    assert b % (NUM_WORKERS * 8) == 0
    bw = b // NUM_WORKERS
    assert bw == 128
    assert d == 64
    assert l % NBUF == 0
    jgroups = bw // LANES

    mesh = plsc.VectorSubcoreMesh(core_axis_name="c", subcore_axis_name="s")

    @functools.partial(
        pl.kernel,
        mesh=mesh,
        compiler_params=pltpu.CompilerParams(needs_layout_passes=False),
        out_type=jax.ShapeDtypeStruct((l, d, b), jnp.float32),
        scratch_types=[
            pltpu.VMEM((NBUF, bw), jnp.int32),
            pltpu.VMEM((NBUF, bw, 2 * d), jnp.float32),
            pltpu.VMEM((NOBUF, d, bw), jnp.float32),
            pltpu.VMEM((l, d), jnp.float32),
            pltpu.VMEM((NBUF, bw), jnp.int32),
            pltpu.VMEM((bw * d,), jnp.float32),
            pltpu.VMEM((bw,), jnp.int32),
            pltpu.SemaphoreType.DMA((NBUF,)),
            pltpu.SemaphoreType.DMA((NBUF,)),
            pltpu.SemaphoreType.DMA((NOBUF,)),
        ],
    )
    def body(xt_hbm, table2_hbm, pe_hbm, out_hbm, idxr, gbuf, obuf, pe_v,
             gidx, skew, cbuf, isem, gsem, ssem):
        cid = lax.axis_index("c")
        sid = lax.axis_index("s")
        wid = sid * NUM_CORES + cid
        wcol = wid * bw

        pltpu.sync_copy(pe_hbm, pe_v)

        iota = lax.iota(jnp.int32, LANES)

        def idx_copy(li, ib):
            return pltpu.make_async_copy(
                xt_hbm.at[li, pl.ds(wcol, bw)], idxr.at[ib], isem.at[ib]
            )

        def prep_and_start_gather(gb):
            for j in range(jgroups):
                vv = idxr[gb, pl.ds(j * LANES, LANES)]
                gidx[gb, pl.ds(j * LANES, LANES)] = lax.shift_right_logical(vv, 1)
            pltpu.make_async_copy(
                table2_hbm.at[gidx.at[gb]], gbuf.at[gb], gsem.at[gb]
            ).start()

        def wait_gather(gb):
            pltpu.make_async_copy(
                table2_hbm.at[gidx.at[gb]], gbuf.at[gb], gsem.at[gb]
            ).wait()

        def store_copy(li, ob):
            return pltpu.make_async_copy(
                obuf.at[ob], out_hbm.at[li, :, pl.ds(wcol, bw)], ssem.at[ob]
            )

        def compute(li, gb, ob):
            iota128 = iota * 128
            iota_g = [iota + (16 * gg) for gg in range(d // LANES)]
            masks = []
            for jg in range(jgroups):
                vv = idxr[gb, pl.ds(jg * LANES, LANES)]
                cbuf[pl.ds(jg * LANES, LANES)] = lax.shift_left(vv & 1, 6)
                masks.append(vv != PAD_IDX)
            zero = jnp.zeros((LANES,), jnp.float32)

            for jg in range(jgroups):
                j0 = jg * LANES

                @plsc.parallel_loop(0, LANES, unroll=2)
                def m_loop(m):
                    j_full = jnp.full((LANES,), j0 + m, jnp.int32)
                    off = plsc.load_gather(cbuf, [j_full])
                    patt = iota128 + ((iota + m) & 15)
                    for gg in range(d // LANES):
                        colv = off + iota_g[gg]
                        g = plsc.load_gather(gbuf.at[gb], [j_full, colv])
                        plsc.store_scatter(
                            skew, [patt + (16 * gg * 128 + j0)], g
                        )

            l_full = jnp.full((LANES,), li, jnp.int32)

            @plsc.parallel_loop(0, d, unroll=2, carry=tuple(masks))
            def f_loop(f, c):
                f_full = jnp.full((LANES,), f, jnp.int32)
                pe_b = plsc.load_gather(pe_v, [l_full, f_full])
                rot = (iota + f) & 15
                base = f * 128
                for jg in range(jgroups):
                    g = plsc.load_gather(skew, [rot + (base + jg * LANES)])
                    g = jnp.where(c[jg], g, zero)
                    obuf[ob, f, pl.ds(jg * LANES, LANES)] = g + pe_b
                return c

        for li0 in range(3):
            idx_copy(jnp.int32(li0), li0).start()
        for li0 in range(2):
            idx_copy(jnp.int32(li0), li0).wait()
            prep_and_start_gather(li0)

        def quad(t, carry):
            for bb in range(NBUF):
                li = t * NBUF + bb
                ob = bb % NOBUF

                @pl.when(li + 3 < l)
                def _refill_idx():
                    idx_copy(li + 3, (bb + 3) % NBUF).start()

                @pl.when(li + 2 < l)
                def _refill_gather():
                    idx_copy(li + 2, (bb + 2) % NBUF).wait()
                    prep_and_start_gather((bb + 2) % NBUF)

                wait_gather(bb)

                @pl.when(li >= NOBUF)
                def _drain():
                    store_copy(li - NOBUF, ob).wait()

                compute(li, bb, ob)
                store_copy(li, ob).start()
            return carry

        lax.fori_loop(0, l // NBUF, quad, 0)

        for li0 in range(l - NOBUF, l):
            store_copy(jnp.int32(li0), li0 % NOBUF).wait()

    return body


def kernel(x, table, pe):
    b, l = x.shape
    v, d = table.shape
    xt = x.T
    table2 = table.reshape(v // 2, 2 * d)
    pe_l = pe[:l]
    lookup = _make_lookup(b, l, d, v)
    out_t = lookup(xt, table2, pe_l)
    return out_t.transpose(2, 0, 1)
